# baseline (speedup 1.0000x reference)
import functools

import jax
import jax.numpy as jnp
from jax import lax
from jax.experimental import pallas as pl
from jax.experimental.pallas import tpu as pltpu
from jax.experimental.pallas import tpu_sc as plsc

_NC = 2
_NS = 16
_NW = _NC * _NS
_CHUNK = 128

_NJ = 26
_NB = 4096
_D = 32
_V = 1000000
_BPW = _NB // _NW * _NJ

_TC_FULL = _V // 128
_TC_TAIL = _V - _TC_FULL * 128


_QC = 4
_QW = _QC * 128
_PITCH = _QW + 5


@functools.lru_cache(maxsize=None)
def _make_detile():
    mesh = plsc.VectorSubcoreMesh(core_axis_name="c", subcore_axis_name="s")
    n_q = _TC_FULL // _QC
    per_w = n_q // _NW
    extra = n_q - per_w * _NW

    @functools.partial(
        pl.kernel,
        mesh=mesh,
        out_type=jax.ShapeDtypeStruct((_V // 4, 128), jnp.float32),
        scratch_types=[
            pltpu.VMEM((2, 32, _PITCH), jnp.float32),
            pltpu.VMEM((2, 32 * _QC, 128), jnp.float32),
            pltpu.SemaphoreType.DMA,
            pltpu.SemaphoreType.DMA,
            pltpu.SemaphoreType.DMA,
            pltpu.SemaphoreType.DMA,
        ],
        compiler_params=pltpu.CompilerParams(
            use_tc_tiling_on_sc=True, needs_layout_passes=False
        ),
    )
    def detile_kernel(wt_hbm, tail_hbm, packed_hbm, vbuf, obuf,
                      si0, si1, so0, so1):
        wid = lax.axis_index("s") * _NC + lax.axis_index("c")
        start = wid * per_w + jnp.minimum(wid, extra)
        count = per_w + jnp.where(wid < extra, 1, 0)
        sis = (si0, si1)
        sos = (so0, so1)

        def fire_in(t, p):
            for d_blk in range(4):
                pltpu.async_copy(
                    wt_hbm.at[pl.ds(d_blk * 8, 8), pl.ds(t * _QW, _QW)],
                    vbuf.at[p, pl.ds(d_blk * 8, 8), pl.ds(0, _QW)],
                    sis[p],
                )

        def wait_in(p):
            for d_blk in range(4):
                pltpu.make_async_copy(
                    wt_hbm.at[pl.ds(0, 8), pl.ds(0, _QW)],
                    vbuf.at[p, pl.ds(d_blk * 8, 8), pl.ds(0, _QW)],
                    sis[p],
                ).wait()

        def transpose(p, q, dlo, dhi, vzero):
            @plsc.parallel_loop(0, 32 * _QC * 8, unroll=8,
                                carry=(dlo, dhi, vzero))
            def _(m, c):
                cdlo, cdhi, cvz = c
                r = m // 8
                g = m - r * 8
                dv = jnp.where((g % 2) == 0, cdlo, cdhi)
                vv = cvz + (4 * r + g // 2)
                vals = plsc.load_gather(vbuf.at[p], [dv, vv])
                obuf[q, r, pl.ds(g * 16, 16)] = vals
                return c

        def fire_out(t, q):
            pltpu.async_copy(
                obuf.at[q], packed_hbm.at[pl.ds(t * 32 * _QC, 32 * _QC)],
                sos[q],
            )

        def wait_out(q):
            pltpu.make_async_copy(
                packed_hbm.at[pl.ds(0, 32 * _QC)], obuf.at[q], sos[q]
            ).wait()

        fire_in(start, 0)
        iota = lax.iota(jnp.int32, 16)
        dhi0 = iota + 16
        vzero0 = iota * 0

        def body2(i, carry):
            dlo, dhi, vzero = carry
            t = start + i * 2

            @pl.when(t + 1 < start + count)
            def _():
                fire_in(t + 1, 1)

            wait_in(0)
            transpose(0, 0, dlo, dhi, vzero)

            @pl.when(i >= 1)
            def _():
                wait_out(0)

            fire_out(t, 0)

            @pl.when(t + 1 < start + count)
            def _():
                @pl.when(t + 2 < start + count)
                def _():
                    fire_in(t + 2, 0)

                wait_in(1)
                transpose(1, 1, dlo, dhi, vzero)

                @pl.when(i >= 1)
                def _():
                    wait_out(1)

                fire_out(t + 1, 1)

            return (dlo, dhi, vzero)

        n2 = (count + 1) // 2
        lax.fori_loop(0, n2, body2, (iota, dhi0, vzero0), unroll=False)
        wait_out(0)

        @pl.when(count >= 2)
        def _():
            wait_out(1)

        @pl.when(wid == _NW - 1)
        def _():
            pltpu.sync_copy(tail_hbm, obuf.at[0, pl.ds(0, 16)])
            pltpu.sync_copy(
                obuf.at[0, pl.ds(0, 16)],
                packed_hbm.at[pl.ds(_TC_FULL * 32, 16)],
            )

    return detile_kernel


@functools.lru_cache(maxsize=None)
def _make_gather():
    n_chunks = _BPW // _CHUNK
    mesh = plsc.VectorSubcoreMesh(core_axis_name="c", subcore_axis_name="s")

    @functools.partial(
        pl.kernel,
        mesh=mesh,
        out_type=jax.ShapeDtypeStruct((_NJ, _D // 8, _NW, 8, 128), jnp.float32),
        scratch_types=[
            pltpu.VMEM((_BPW,), jnp.int32),
            pltpu.VMEM((_BPW, _D), jnp.float32),
            pltpu.VMEM((2, _D // 8, 8, 128), jnp.float32),
            pltpu.SemaphoreType.DMA,
            pltpu.SemaphoreType.DMA,
            pltpu.SemaphoreType.DMA,
        ],
        compiler_params=pltpu.CompilerParams(
            use_tc_tiling_on_sc=False, needs_layout_passes=False
        ),
    )
    def gather_kernel(table_hbm, idx_hbm, out_hbm, idx_v, rows_v, buf_v,
                      sem_g, sem_o0, sem_o1):
        wid = lax.axis_index("s") * _NC + lax.axis_index("c")
        base = wid * _BPW
        pltpu.sync_copy(idx_hbm.at[pl.ds(base, _BPW)], idx_v)

        def fire(i, carry):
            off = i * _CHUNK
            pltpu.async_copy(
                table_hbm.at[idx_v.at[pl.ds(off, _CHUNK)]],
                rows_v.at[pl.ds(off, _CHUNK)],
                sem_g,
            )
            return carry

        lax.fori_loop(0, n_chunks, fire, 0)
        pltpu.make_async_copy(
            table_hbm.at[pl.ds(0, _BPW)], rows_v, sem_g
        ).wait()

        sems = (sem_o0, sem_o1)

        def transpose_j(i, j, p, ib, czero):
            sem = sems[p]

            @pl.when(i >= 1)
            def _():
                for d_blk in range(_D // 8):
                    pltpu.make_async_copy(
                        buf_v.at[p, d_blk],
                        out_hbm.at[0, d_blk, wid],
                        sem,
                    ).wait()

            @plsc.parallel_loop(0, 8 * _D, unroll=8, carry=(ib, czero))
            def _(m, c):
                cib, ccz = c
                k = m // _D
                d = m - k * _D
                row_k = cib + (k * 16 * _NJ + j)
                vals = plsc.load_gather(rows_v, [row_k, ccz + d])
                dblk = d // 8
                buf_v[p, dblk, d - dblk * 8, pl.ds(k * 16, 16)] = vals
                return c
            for d_blk in range(_D // 8):
                pltpu.async_copy(
                    buf_v.at[p, d_blk], out_hbm.at[j, d_blk, wid], sem
                )

        ib0 = lax.iota(jnp.int32, 16) * _NJ
        cz0 = lax.iota(jnp.int32, 16) * 0

        def body2(i, carry):
            ib, cz = carry
            j = i * 2
            transpose_j(i, j, 0, ib, cz)
            transpose_j(i, j + 1, 1, ib, cz)
            return (ib, cz)

        lax.fori_loop(0, _NJ // 2, body2, (ib0, cz0))
        for p in range(2):
            for d_blk in range(_D // 8):
                pltpu.make_async_copy(
                    buf_v.at[p, d_blk], out_hbm.at[0, d_blk, wid], sems[p]
                ).wait()

    return gather_kernel


def kernel(x, weight):
    tail = weight[_TC_FULL * 128:].reshape(_TC_TAIL // 4, 128)
    packed = _make_detile()(weight.T, tail)
    table_lin = packed.reshape(_V, _D)
    idx = x.reshape(-1).astype(jnp.int32)
    out5 = jnp.zeros((_NJ, _D // 8, _NW, 8, 128), jnp.float32) + table_lin[0, 0] + idx[0]
    return out5.transpose(2, 4, 0, 1, 3).reshape(_NB, _NJ, _D)


# speedup vs baseline: 1.4958x; 1.4958x over previous
import functools

import jax
import jax.numpy as jnp
from jax import lax
from jax.experimental import pallas as pl
from jax.experimental.pallas import tpu as pltpu
from jax.experimental.pallas import tpu_sc as plsc

_NC = 2
_NS = 16
_NW = _NC * _NS
_CHUNK = 128

_NJ = 26
_NB = 4096
_D = 32
_V = 1000000
_BPW = _NB // _NW * _NJ

_TC_FULL = _V // 128
_TC_TAIL = _V - _TC_FULL * 128


_QC = 4
_QW = _QC * 128
_PITCH = _QW + 5


@functools.lru_cache(maxsize=None)
def _make_detile():
    mesh = plsc.VectorSubcoreMesh(core_axis_name="c", subcore_axis_name="s")
    n_q = _TC_FULL // _QC
    per_w = n_q // _NW
    extra = n_q - per_w * _NW

    @functools.partial(
        pl.kernel,
        mesh=mesh,
        out_type=jax.ShapeDtypeStruct((_V // 4, 128), jnp.float32),
        scratch_types=[
            pltpu.VMEM((2, 32, _PITCH), jnp.float32),
            pltpu.VMEM((2, 32 * _QC, 128), jnp.float32),
            pltpu.SemaphoreType.DMA,
            pltpu.SemaphoreType.DMA,
            pltpu.SemaphoreType.DMA,
            pltpu.SemaphoreType.DMA,
        ],
        compiler_params=pltpu.CompilerParams(
            use_tc_tiling_on_sc=True, needs_layout_passes=False
        ),
    )
    def detile_kernel(wt_hbm, tail_hbm, packed_hbm, vbuf, obuf,
                      si0, si1, so0, so1):
        wid = lax.axis_index("s") * _NC + lax.axis_index("c")
        start = wid * per_w + jnp.minimum(wid, extra)
        count = per_w + jnp.where(wid < extra, 1, 0)
        sis = (si0, si1)
        sos = (so0, so1)

        def fire_in(t, p):
            for d_blk in range(4):
                pltpu.async_copy(
                    wt_hbm.at[pl.ds(d_blk * 8, 8), pl.ds(t * _QW, _QW)],
                    vbuf.at[p, pl.ds(d_blk * 8, 8), pl.ds(0, _QW)],
                    sis[p],
                )

        def wait_in(p):
            for d_blk in range(4):
                pltpu.make_async_copy(
                    wt_hbm.at[pl.ds(0, 8), pl.ds(0, _QW)],
                    vbuf.at[p, pl.ds(d_blk * 8, 8), pl.ds(0, _QW)],
                    sis[p],
                ).wait()

        def transpose(p, q, dlo, dhi, vzero):
            @plsc.parallel_loop(0, 32 * _QC * 8, unroll=8,
                                carry=(dlo, dhi, vzero))
            def _(m, c):
                cdlo, cdhi, cvz = c
                r = m // 8
                g = m - r * 8
                dv = jnp.where((g % 2) == 0, cdlo, cdhi)
                vv = cvz + (4 * r + g // 2)
                vals = plsc.load_gather(vbuf.at[p], [dv, vv])
                obuf[q, r, pl.ds(g * 16, 16)] = vals
                return c

        def fire_out(t, q):
            pltpu.async_copy(
                obuf.at[q], packed_hbm.at[pl.ds(t * 32 * _QC, 32 * _QC)],
                sos[q],
            )

        def wait_out(q):
            pltpu.make_async_copy(
                packed_hbm.at[pl.ds(0, 32 * _QC)], obuf.at[q], sos[q]
            ).wait()

        fire_in(start, 0)
        iota = lax.iota(jnp.int32, 16)
        dhi0 = iota + 16
        vzero0 = iota * 0

        def body2(i, carry):
            dlo, dhi, vzero = carry
            t = start + i * 2

            @pl.when(t + 1 < start + count)
            def _():
                fire_in(t + 1, 1)

            wait_in(0)
            transpose(0, 0, dlo, dhi, vzero)

            @pl.when(i >= 1)
            def _():
                wait_out(0)

            fire_out(t, 0)

            @pl.when(t + 1 < start + count)
            def _():
                @pl.when(t + 2 < start + count)
                def _():
                    fire_in(t + 2, 0)

                wait_in(1)
                transpose(1, 1, dlo, dhi, vzero)

                @pl.when(i >= 1)
                def _():
                    wait_out(1)

                fire_out(t + 1, 1)

            return (dlo, dhi, vzero)

        n2 = (count + 1) // 2
        lax.fori_loop(0, n2, body2, (iota, dhi0, vzero0), unroll=False)
        wait_out(0)

        @pl.when(count >= 2)
        def _():
            wait_out(1)

        @pl.when(wid == _NW - 1)
        def _():
            pltpu.sync_copy(tail_hbm, obuf.at[0, pl.ds(0, 16)])
            pltpu.sync_copy(
                obuf.at[0, pl.ds(0, 16)],
                packed_hbm.at[pl.ds(_TC_FULL * 32, 16)],
            )

    return detile_kernel


@functools.lru_cache(maxsize=None)
def _make_gather():
    n_chunks = _BPW // _CHUNK
    mesh = plsc.VectorSubcoreMesh(core_axis_name="c", subcore_axis_name="s")

    @functools.partial(
        pl.kernel,
        mesh=mesh,
        out_type=jax.ShapeDtypeStruct((_NJ, _D // 8, _NW, 8, 128), jnp.float32),
        scratch_types=[
            pltpu.VMEM((_BPW,), jnp.int32),
            pltpu.VMEM((_BPW, _D), jnp.float32),
            pltpu.VMEM((2, _D // 8, 8, 128), jnp.float32),
            pltpu.SemaphoreType.DMA,
            pltpu.SemaphoreType.DMA,
            pltpu.SemaphoreType.DMA,
        ],
        compiler_params=pltpu.CompilerParams(
            use_tc_tiling_on_sc=False, needs_layout_passes=False
        ),
    )
    def gather_kernel(table_hbm, idx_hbm, out_hbm, idx_v, rows_v, buf_v,
                      sem_g, sem_o0, sem_o1):
        wid = lax.axis_index("s") * _NC + lax.axis_index("c")
        base = wid * _BPW
        pltpu.sync_copy(idx_hbm.at[pl.ds(base, _BPW)], idx_v)

        def fire(i, carry):
            off = i * _CHUNK
            pltpu.async_copy(
                table_hbm.at[idx_v.at[pl.ds(off, _CHUNK)]],
                rows_v.at[pl.ds(off, _CHUNK)],
                sem_g,
            )
            return carry

        lax.fori_loop(0, n_chunks, fire, 0)
        pltpu.make_async_copy(
            table_hbm.at[pl.ds(0, _BPW)], rows_v, sem_g
        ).wait()

        sems = (sem_o0, sem_o1)

        def transpose_j(i, j, p, ib, czero):
            sem = sems[p]

            @pl.when(i >= 1)
            def _():
                for d_blk in range(_D // 8):
                    pltpu.make_async_copy(
                        buf_v.at[p, d_blk],
                        out_hbm.at[0, d_blk, wid],
                        sem,
                    ).wait()

            @plsc.parallel_loop(0, 8 * _D, unroll=8, carry=(ib, czero))
            def _(m, c):
                cib, ccz = c
                k = m // _D
                d = m - k * _D
                row_k = cib + (k * 16 * _NJ + j)
                vals = plsc.load_gather(rows_v, [row_k, ccz + d])
                dblk = d // 8
                buf_v[p, dblk, d - dblk * 8, pl.ds(k * 16, 16)] = vals
                return c
            for d_blk in range(_D // 8):
                pltpu.async_copy(
                    buf_v.at[p, d_blk], out_hbm.at[j, d_blk, wid], sem
                )

        ib0 = lax.iota(jnp.int32, 16) * _NJ
        cz0 = lax.iota(jnp.int32, 16) * 0

        def body2(i, carry):
            ib, cz = carry
            j = i * 2
            transpose_j(i, j, 0, ib, cz)
            transpose_j(i, j + 1, 1, ib, cz)
            return (ib, cz)

        lax.fori_loop(0, _NJ // 2, body2, (ib0, cz0))
        for p in range(2):
            for d_blk in range(_D // 8):
                pltpu.make_async_copy(
                    buf_v.at[p, d_blk], out_hbm.at[0, d_blk, wid], sems[p]
                ).wait()

    return gather_kernel


def kernel(x, weight):
    tail = weight[_TC_FULL * 128:].reshape(_TC_TAIL // 4, 128)
    packed = _make_detile()(weight.T, tail)
    table_lin = packed.reshape(_V, _D)
    idx = x.reshape(-1).astype(jnp.int32)
    out5 = _make_gather()(table_lin, idx)
    return out5.transpose(2, 4, 0, 1, 3).reshape(_NB, _NJ, _D)


# speedup vs baseline: 1.6084x; 1.0753x over previous
import functools

import jax
import jax.numpy as jnp
from jax import lax
from jax.experimental import pallas as pl
from jax.experimental.pallas import tpu as pltpu
from jax.experimental.pallas import tpu_sc as plsc

_NC = 2
_NS = 16
_NW = _NC * _NS
_CHUNK = 128

_NJ = 26
_NB = 4096
_D = 32
_V = 1000000
_BPW = _NB // _NW * _NJ

_TC_FULL = _V // 128
_TC_TAIL = _V - _TC_FULL * 128


_QC = 4
_QW = _QC * 128
_PITCH = _QW + 5


@functools.lru_cache(maxsize=None)
def _make_detile():
    mesh = plsc.VectorSubcoreMesh(core_axis_name="c", subcore_axis_name="s")
    n_q = _TC_FULL // _QC
    per_w = n_q // _NW
    extra = n_q - per_w * _NW

    @functools.partial(
        pl.kernel,
        mesh=mesh,
        out_type=jax.ShapeDtypeStruct((_V // 4, 128), jnp.float32),
        scratch_types=[
            pltpu.VMEM((2, 32, _PITCH), jnp.float32),
            pltpu.VMEM((2, 32 * _QC, 128), jnp.float32),
            pltpu.SemaphoreType.DMA,
            pltpu.SemaphoreType.DMA,
            pltpu.SemaphoreType.DMA,
            pltpu.SemaphoreType.DMA,
        ],
        compiler_params=pltpu.CompilerParams(
            use_tc_tiling_on_sc=True, needs_layout_passes=False
        ),
    )
    def detile_kernel(wt_hbm, tail_hbm, packed_hbm, vbuf, obuf,
                      si0, si1, so0, so1):
        wid = lax.axis_index("s") * _NC + lax.axis_index("c")
        start = wid * per_w + jnp.minimum(wid, extra)
        count = per_w + jnp.where(wid < extra, 1, 0)
        sis = (si0, si1)
        sos = (so0, so1)

        def fire_in(t, p):
            for d_blk in range(4):
                pltpu.async_copy(
                    wt_hbm.at[pl.ds(d_blk * 8, 8), pl.ds(t * _QW, _QW)],
                    vbuf.at[p, pl.ds(d_blk * 8, 8), pl.ds(0, _QW)],
                    sis[p],
                )

        def wait_in(p):
            for d_blk in range(4):
                pltpu.make_async_copy(
                    wt_hbm.at[pl.ds(0, 8), pl.ds(0, _QW)],
                    vbuf.at[p, pl.ds(d_blk * 8, 8), pl.ds(0, _QW)],
                    sis[p],
                ).wait()

        def transpose(p, q, dlo, dhi, vzero):
            for half, dvec in ((0, dlo), (1, dhi)):
                @plsc.parallel_loop(0, 32 * _QC * 4, unroll=16,
                                    carry=(dvec, vzero))
                def _(m, c):
                    dv, cvz = c
                    r = m // 4
                    h = m - r * 4
                    vv = cvz + (4 * r + h)
                    vals = plsc.load_gather(vbuf.at[p], [dv, vv])
                    obuf[q, r, pl.ds(h * 32 + half * 16, 16)] = vals
                    return c

        def fire_out(t, q):
            pltpu.async_copy(
                obuf.at[q], packed_hbm.at[pl.ds(t * 32 * _QC, 32 * _QC)],
                sos[q],
            )

        def wait_out(q):
            pltpu.make_async_copy(
                packed_hbm.at[pl.ds(0, 32 * _QC)], obuf.at[q], sos[q]
            ).wait()

        fire_in(start, 0)
        iota = lax.iota(jnp.int32, 16)
        dhi0 = iota + 16
        vzero0 = iota * 0

        def body2(i, carry):
            dlo, dhi, vzero = carry
            t = start + i * 2

            @pl.when(t + 1 < start + count)
            def _():
                fire_in(t + 1, 1)

            wait_in(0)
            transpose(0, 0, dlo, dhi, vzero)

            @pl.when(i >= 1)
            def _():
                wait_out(0)

            fire_out(t, 0)

            @pl.when(t + 1 < start + count)
            def _():
                @pl.when(t + 2 < start + count)
                def _():
                    fire_in(t + 2, 0)

                wait_in(1)
                transpose(1, 1, dlo, dhi, vzero)

                @pl.when(i >= 1)
                def _():
                    wait_out(1)

                fire_out(t + 1, 1)

            return (dlo, dhi, vzero)

        n2 = (count + 1) // 2
        lax.fori_loop(0, n2, body2, (iota, dhi0, vzero0), unroll=False)
        wait_out(0)

        @pl.when(count >= 2)
        def _():
            wait_out(1)

        @pl.when(wid == _NW - 1)
        def _():
            pltpu.sync_copy(tail_hbm, obuf.at[0, pl.ds(0, 16)])
            pltpu.sync_copy(
                obuf.at[0, pl.ds(0, 16)],
                packed_hbm.at[pl.ds(_TC_FULL * 32, 16)],
            )

    return detile_kernel


@functools.lru_cache(maxsize=None)
def _make_gather():
    n_chunks = _BPW // _CHUNK
    mesh = plsc.VectorSubcoreMesh(core_axis_name="c", subcore_axis_name="s")

    @functools.partial(
        pl.kernel,
        mesh=mesh,
        out_type=jax.ShapeDtypeStruct((_NJ, _D // 8, _NW, 8, 128), jnp.float32),
        scratch_types=[
            pltpu.VMEM((_BPW,), jnp.int32),
            pltpu.VMEM((_BPW, _D), jnp.float32),
            pltpu.VMEM((2, _D // 8, 8, 128), jnp.float32),
            pltpu.SemaphoreType.DMA,
            pltpu.SemaphoreType.DMA,
            pltpu.SemaphoreType.DMA,
        ],
        compiler_params=pltpu.CompilerParams(
            use_tc_tiling_on_sc=False, needs_layout_passes=False
        ),
    )
    def gather_kernel(table_hbm, idx_hbm, out_hbm, idx_v, rows_v, buf_v,
                      sem_g, sem_o0, sem_o1):
        wid = lax.axis_index("s") * _NC + lax.axis_index("c")
        base = wid * _BPW
        pltpu.sync_copy(idx_hbm.at[pl.ds(base, _BPW)], idx_v)

        def fire(i, carry):
            off = i * _CHUNK
            pltpu.async_copy(
                table_hbm.at[idx_v.at[pl.ds(off, _CHUNK)]],
                rows_v.at[pl.ds(off, _CHUNK)],
                sem_g,
            )
            return carry

        lax.fori_loop(0, n_chunks, fire, 0)
        pltpu.make_async_copy(
            table_hbm.at[pl.ds(0, _BPW)], rows_v, sem_g
        ).wait()

        sems = (sem_o0, sem_o1)

        def transpose_j(i, j, p, ib, czero):
            sem = sems[p]

            @pl.when(i >= 1)
            def _():
                for d_blk in range(_D // 8):
                    pltpu.make_async_copy(
                        buf_v.at[p, d_blk],
                        out_hbm.at[0, d_blk, wid],
                        sem,
                    ).wait()

            @plsc.parallel_loop(0, 8 * _D, unroll=8, carry=(ib, czero))
            def _(m, c):
                cib, ccz = c
                k = m // _D
                d = m - k * _D
                row_k = cib + (k * 16 * _NJ + j)
                vals = plsc.load_gather(rows_v, [row_k, ccz + d])
                dblk = d // 8
                buf_v[p, dblk, d - dblk * 8, pl.ds(k * 16, 16)] = vals
                return c
            for d_blk in range(_D // 8):
                pltpu.async_copy(
                    buf_v.at[p, d_blk], out_hbm.at[j, d_blk, wid], sem
                )

        ib0 = lax.iota(jnp.int32, 16) * _NJ
        cz0 = lax.iota(jnp.int32, 16) * 0

        def body2(i, carry):
            ib, cz = carry
            j = i * 2
            transpose_j(i, j, 0, ib, cz)
            transpose_j(i, j + 1, 1, ib, cz)
            return (ib, cz)

        lax.fori_loop(0, _NJ // 2, body2, (ib0, cz0))
        for p in range(2):
            for d_blk in range(_D // 8):
                pltpu.make_async_copy(
                    buf_v.at[p, d_blk], out_hbm.at[0, d_blk, wid], sems[p]
                ).wait()

    return gather_kernel


def kernel(x, weight):
    tail = weight[_TC_FULL * 128:].reshape(_TC_TAIL // 4, 128)
    packed = _make_detile()(weight.T, tail)
    table_lin = packed.reshape(_V, _D)
    idx = x.reshape(-1).astype(jnp.int32)
    out5 = _make_gather()(table_lin, idx)
    return out5.transpose(2, 4, 0, 1, 3).reshape(_NB, _NJ, _D)
